# baseline (speedup 1.0000x reference)
import functools

import jax
import jax.numpy as jnp
from jax import lax
from jax.experimental import pallas as pl
from jax.experimental.pallas import tpu as pltpu
from jax.experimental.pallas import tpu_sc as plsc

VOCAB = 1000000
HIDDEN = 64
BATCH = 16384
HIST = 50
B_TOTAL = BATCH * HIST

NUM_CORES = 2
NUM_SUBCORES = 16
NW = NUM_CORES * NUM_SUBCORES
B_PER_W = B_TOTAL // NW
CHUNK = 512
N_CHUNKS = B_PER_W // CHUNK


def _make_gather():
    mesh = plsc.VectorSubcoreMesh(core_axis_name="c", subcore_axis_name="s")

    @functools.partial(
        pl.kernel,
        mesh=mesh,
        out_type=jax.ShapeDtypeStruct((B_TOTAL, HIDDEN), jnp.float32),
        scratch_types=[
            pltpu.VMEM((CHUNK,), jnp.int32),
            pltpu.VMEM((CHUNK, HIDDEN), jnp.float32),
            pltpu.SemaphoreType.DMA,
        ],
        compiler_params=pltpu.CompilerParams(use_tc_tiling_on_sc=False),
    )
    def gather_kernel(idx_hbm, table_hbm, out_hbm, idx_v, rows_v, sem):
        wid = lax.axis_index("s") * NUM_CORES + lax.axis_index("c")
        base = wid * B_PER_W

        def body(i, _):
            off = base + i * CHUNK
            pltpu.sync_copy(idx_hbm.at[pl.ds(off, CHUNK)], idx_v)
            pltpu.async_copy(table_hbm.at[idx_v], rows_v, sem).wait()
            pltpu.sync_copy(rows_v, out_hbm.at[pl.ds(off, CHUNK)])
            return 0

        lax.fori_loop(0, N_CHUNKS, body, 0)

    return gather_kernel


_gather = _make_gather()


@jax.jit
def kernel(tokens, embedding):
    idx = tokens.astype(jnp.int32).reshape(B_TOTAL)
    out = _gather(idx, embedding)
    return out.reshape(BATCH, HIST, HIDDEN)


# speedup vs baseline: 1.0426x; 1.0426x over previous
import functools

import jax
import jax.numpy as jnp
from jax import lax
from jax.experimental import pallas as pl
from jax.experimental.pallas import tpu as pltpu
from jax.experimental.pallas import tpu_sc as plsc

VOCAB = 1000000
HIDDEN = 64
BATCH = 16384
HIST = 50
B_TOTAL = BATCH * HIST

NUM_CORES = 2
NUM_SUBCORES = 16
NW = NUM_CORES * NUM_SUBCORES
B_PER_W = B_TOTAL // NW
CHUNK = 256
NBUF = 4
N_CHUNKS = B_PER_W // CHUNK
N_GROUPS = N_CHUNKS // NBUF
assert N_CHUNKS % NBUF == 0


def _make_gather():
    mesh = plsc.VectorSubcoreMesh(core_axis_name="c", subcore_axis_name="s")

    @functools.partial(
        pl.kernel,
        mesh=mesh,
        out_type=jax.ShapeDtypeStruct((B_TOTAL, HIDDEN), jnp.float32),
        scratch_types=[
            pltpu.VMEM((B_PER_W,), jnp.int32),
            *[pltpu.VMEM((CHUNK, HIDDEN), jnp.float32) for _ in range(NBUF)],
            *[pltpu.SemaphoreType.DMA for _ in range(2 * NBUF)],
        ],
        compiler_params=pltpu.CompilerParams(use_tc_tiling_on_sc=False),
    )
    def gather_kernel(idx_hbm, table_hbm, out_hbm, idx_v, *bufs):
        rows = bufs[:NBUF]
        sg = bufs[NBUF : 2 * NBUF]
        ss = bufs[2 * NBUF : 3 * NBUF]
        wid = lax.axis_index("s") * NUM_CORES + lax.axis_index("c")
        base = wid * B_PER_W

        pltpu.sync_copy(idx_hbm.at[pl.ds(base, B_PER_W)], idx_v)

        def g_desc(i, b):
            src = table_hbm.at[idx_v.at[pl.ds(i * CHUNK, CHUNK)]]
            return pltpu.make_async_copy(src, rows[b], sg[b])

        def s_desc(i, b):
            dst = out_hbm.at[pl.ds(base + i * CHUNK, CHUNK)]
            return pltpu.make_async_copy(rows[b], dst, ss[b])

        g_desc(0, 0).start()
        for b in range(1, NBUF):
            g_desc(b, b).start()
            g_desc(b - 1, b - 1).wait()
            s_desc(b - 1, b - 1).start()

        def group(g, _):
            i0 = g * NBUF
            for b in range(NBUF):
                i = i0 + b
                bp = (b - 1) % NBUF
                s_desc(i - NBUF, b).wait()
                g_desc(i, b).start()
                g_desc(i - 1, bp).wait()
                s_desc(i - 1, bp).start()
            return 0

        lax.fori_loop(1, N_GROUPS, group, 0)

        last = N_CHUNKS - 1
        bl = last % NBUF
        g_desc(last, bl).wait()
        s_desc(last, bl).start()
        for b in range(NBUF):
            s_desc(N_CHUNKS - NBUF + b, b).wait()

    return gather_kernel


_gather = _make_gather()


@jax.jit
def kernel(tokens, embedding):
    idx = tokens.astype(jnp.int32).reshape(B_TOTAL)
    out = _gather(idx, embedding)
    return out.reshape(BATCH, HIST, HIDDEN)
